# Initial kernel scaffold; baseline (speedup 1.0000x reference)
import jax
import jax.numpy as jnp
from jax.experimental import pallas as pl


def kernel(atom_feats, bond_feats, global_feats, bond_atom, atom_graph, bond_graph, atom_w0, atom_b0, atom_w1, atom_b1, atom_w2, atom_b2, bond_w0, bond_b0, bond_w1, bond_b1, bond_w2, bond_b2, glob_w0, glob_b0, glob_w1, glob_b1, glob_w2, glob_b2):
    raise NotImplementedError("write your pallas kernel here")


# speedup vs baseline: 2.5791x; 2.5791x over previous
import functools

import jax
import jax.numpy as jnp
from jax import lax
from jax.experimental import pallas as pl
from jax.experimental.pallas import tpu as pltpu
from jax.experimental.pallas import tpu_sc as plsc

DA = 50000
DB = 200000
DG = 256
D = 128

NC, NS = 2, 16
NW = NC * NS

R = 8448
NPASS = 3
SUM_PAD = 6 * R
ACC_ROWS = R + 16
CNT_ROWS = R + 64
EP_T = 25024
EPAD = NS * EP_T
ECH = 1472
DR = 64

BW_T = 6272
BPAD = NW * BW_T
GCH = BW_T // 128

@functools.cache
def _sc_mesh():
    return plsc.VectorSubcoreMesh(core_axis_name="c", subcore_axis_name="s",
                                  num_cores=NC, num_subcores=NS)


def _scatter_body_maker(with_gather):
    def body(e_hbm, bond_hbm, sum_hbm,
             acc, e_stage, loc2d, bid2d, zrows, gbuf, sem):
        c = lax.axis_index("c")
        s = lax.axis_index("s")
        iota16 = lax.iota(jnp.int32, 16)
        zeros16 = jnp.zeros((16,), jnp.float32)
        ones16 = jnp.ones((16,), jnp.float32)

        def init_body(t, carry):
            zrows[t >> 3, pl.ds((t & 7) * 16, 16)] = zeros16
            if not with_gather:
                gbuf[t >> 3, pl.ds((t & 7) * 16, 16)] = ones16
            return carry
        lax.fori_loop(0, DR * 8, init_body, 0)

        def drain(rr):
            if with_gather:
                pltpu.async_copy(bond_hbm.at[bid2d.at[rr]], gbuf, sem).wait()
            pltpu.sync_copy(gbuf, acc.at[loc2d.at[rr]], add=True)

        for p in range(NPASS):
            r = c * NPASS + p
            lo = r * R

            a0 = s * (ACC_ROWS // 16)
            for k in range(8):
                pltpu.sync_copy(zrows, acc.at[pl.ds(a0 + k * DR, DR)])
            pltpu.sync_copy(
                zrows.at[pl.ds(0, ACC_ROWS // 16 - 8 * DR)],
                acc.at[pl.ds(a0 + 8 * DR, ACC_ROWS // 16 - 8 * DR)])
            plsc.subcore_barrier()

            def chunk_body(cb, coff):
                ebase = s * EP_T + cb * ECH
                pltpu.sync_copy(e_hbm.at[pl.ds(ebase, ECH)], e_stage)

                def scan_body(i, off):
                    e = e_stage[pl.ds(i * 16, 16)]
                    inb = (e >= lo) & (e < lo + R)
                    csum = jnp.cumsum(jnp.where(inb, 1, 0).astype(jnp.int32))
                    pos = off + csum - 1
                    prow = lax.shift_right_logical(pos, 6) & 1
                    pcol = pos & (DR - 1)
                    gidx = ebase + i * 16 + iota16
                    plsc.store_scatter(loc2d, [prow, pcol], e - lo, mask=inb)
                    plsc.store_scatter(bid2d, [prow, pcol],
                                       lax.shift_right_logical(gidx, 1),
                                       mask=inb)
                    noff = off + jnp.max(csum)

                    @pl.when(lax.shift_right_logical(noff, 6)
                             > lax.shift_right_logical(off, 6))
                    def _():
                        drain(lax.shift_right_logical(off, 6) & 1)
                    return noff
                return lax.fori_loop(0, ECH // 16, scan_body, coff)
            off = lax.fori_loop(0, EP_T // ECH, chunk_body, jnp.int32(0))

            rem = off & (DR - 1)
            rowi = lax.shift_right_logical(off, 6) & 1

            @pl.when(rem > 0)
            def _():
                for j in range(4):
                    lane = j * 16 + iota16
                    m = lane >= rem
                    plsc.store_scatter(loc2d, [lane * 0 + rowi, lane],
                                       R + (iota16 & 7), mask=m)
                    plsc.store_scatter(
                        bid2d, [lane * 0 + rowi, lane],
                        ((s * 12347 + j * 1031) + iota16 * 997) % DB, mask=m)
                drain(rowi)
            plsc.subcore_barrier()

            w0 = s * (R // 16)
            g0 = r * R + s * (R // 16)
            for k in range(4):
                pltpu.sync_copy(acc.at[pl.ds(w0 + k * 128, 128)],
                                sum_hbm.at[pl.ds(g0 + k * 128, 128)])
            pltpu.sync_copy(acc.at[pl.ds(w0 + 512, R // 16 - 512)],
                            sum_hbm.at[pl.ds(g0 + 512, R // 16 - 512)])
            plsc.subcore_barrier()
    return body


def _make_scatter(with_gather):
    return pl.kernel(
        _scatter_body_maker(with_gather),
        out_type=jax.ShapeDtypeStruct((SUM_PAD, D), jnp.float32),
        mesh=_sc_mesh(),
        scratch_types=(
            pltpu.VMEM_SHARED((ACC_ROWS, D), jnp.float32),
            pltpu.VMEM((ECH,), jnp.int32),
            pltpu.VMEM((2, DR), jnp.int32),
            pltpu.VMEM((2, DR), jnp.int32),
            pltpu.VMEM((DR, D), jnp.float32),
            pltpu.VMEM((DR, D), jnp.float32),
            pltpu.SemaphoreType.DMA,
        ),
        compiler_params=pltpu.CompilerParams(needs_layout_passes=False),
    )


def _scatter_sum(e, bond_feats):
    return _make_scatter(True)(e, bond_feats)


def _scatter_cnt(e, dummy):
    return _make_scatter(False)(e, dummy)


def _gather_pair(src0, src1, p0, p1):
    k = pl.kernel(
        _gather_body,
        out_type=jax.ShapeDtypeStruct((BPAD, D), jnp.float32),
        mesh=_sc_mesh(),
        scratch_types=(
            pltpu.VMEM((128,), jnp.int32),
            pltpu.VMEM((128,), jnp.int32),
            pltpu.VMEM((128, D), jnp.float32),
            pltpu.VMEM((128, D), jnp.float32),
            pltpu.SemaphoreType.DMA,
            pltpu.SemaphoreType.DMA,
        ),
        compiler_params=pltpu.CompilerParams(needs_layout_passes=False),
    )
    return k(src0, src1, p0, p1)


def _gather_body(src0_hbm, src1_hbm, p0_hbm, p1_hbm, gs_hbm,
                 idx0, idx1, buf0, buf1, sem0, sem1):
    c = lax.axis_index("c")
    s = lax.axis_index("s")
    base = (s * NC + c) * BW_T

    def body(cc, carry):
        off = base + cc * 128
        pltpu.sync_copy(src0_hbm.at[pl.ds(off, 128)], idx0)
        pltpu.sync_copy(src1_hbm.at[pl.ds(off, 128)], idx1)
        cp0 = pltpu.async_copy(p0_hbm.at[idx0], buf0, sem0)
        cp1 = pltpu.async_copy(p1_hbm.at[idx1], buf1, sem1)
        cp0.wait()
        cp1.wait()

        def add_body(j, acarry):
            for k in range(8):
                plsc.addupdate(buf0.at[j, pl.ds(k * 16, 16)],
                               buf1[j, pl.ds(k * 16, 16)])
            return acarry
        lax.fori_loop(0, 128, add_body, 0)
        pltpu.sync_copy(buf0, gs_hbm.at[pl.ds(off, 128)])
        return carry
    lax.fori_loop(0, GCH, body, 0)


BA = 1000
BB = 1000


def _softplus(x):
    return jnp.maximum(x, 0.0) + jnp.log1p(jnp.exp(-jnp.abs(x)))


def _dot(a, b):
    return jnp.dot(a, b, preferred_element_type=jnp.float32)


def _pre_tc(gf_ref, a0g_ref, wg_ref, gga_ref, ggb_ref):
    gga_ref[...] = _dot(gf_ref[...], a0g_ref[...])
    ggb_ref[...] = _dot(gf_ref[...], wg_ref[...])


def _atom_tc(af_ref, sb_ref, cnt_ref, ag_ref, a0a_ref, a0b_ref, gga_ref,
             b0_ref, w1_ref, b1_ref, w2_ref, b2_ref, wa0_ref, wa1_ref,
             ao_ref, p0_ref, p1_ref, seg_ref, cntg_ref):
    i = pl.program_id(0)
    cnt = cnt_ref[...][:, 0:1]
    inv = 1.0 / jnp.maximum(cnt, 1.0)
    mb = sb_ref[...] * inv
    ag = ag_ref[...][:, 0:1]
    gio = lax.broadcasted_iota(jnp.int32, (BA, DG), 1)
    oh = (gio == ag).astype(jnp.float32)
    pre = (_dot(af_ref[...], a0a_ref[...]) + _dot(mb, a0b_ref[...])
           + _dot(oh, gga_ref[...]) + b0_ref[...])
    h = _softplus(pre)
    h = _softplus(_dot(h, w1_ref[...]) + b1_ref[...])
    ao = _dot(h, w2_ref[...]) + b2_ref[...]
    ao_ref[...] = ao
    p0_ref[...] = _dot(ao, wa0_ref[...])
    p1_ref[...] = _dot(ao, wa1_ref[...])
    seg = lax.dot_general(oh, ao, (((0,), (0,)), ((), ())),
                          preferred_element_type=jnp.float32)
    cg = jnp.broadcast_to(jnp.sum(oh, axis=0)[:, None], (DG, D))

    @pl.when(i == 0)
    def _():
        seg_ref[...] = seg
        cntg_ref[...] = cg

    @pl.when(i != 0)
    def _():
        seg_ref[...] += seg
        cntg_ref[...] += cg


def _bond_tc(bf_ref, gs_ref, bg_ref, wb_ref, ggb_ref, b0_ref, w1_ref, b1_ref,
             w2_ref, b2_ref, bo_ref, seg_ref, cntg_ref):
    i = pl.program_id(0)
    bg = bg_ref[...][:, 0:1]
    gio = lax.broadcasted_iota(jnp.int32, (BB, DG), 1)
    oh = (gio == bg).astype(jnp.float32)
    pre = (_dot(bf_ref[...], wb_ref[...]) + gs_ref[...]
           + _dot(oh, ggb_ref[...]) + b0_ref[...])
    h = _softplus(pre)
    h = _softplus(_dot(h, w1_ref[...]) + b1_ref[...])
    bo = _dot(h, w2_ref[...]) + b2_ref[...]
    bo_ref[...] = bo
    seg = lax.dot_general(oh, bo, (((0,), (0,)), ((), ())),
                          preferred_element_type=jnp.float32)
    cg = jnp.broadcast_to(jnp.sum(oh, axis=0)[:, None], (DG, D))

    @pl.when(i == 0)
    def _():
        seg_ref[...] = seg
        cntg_ref[...] = cg

    @pl.when(i != 0)
    def _():
        seg_ref[...] += seg
        cntg_ref[...] += cg


def _glob_tc(gf_ref, aseg_ref, acnt_ref, bseg_ref, bcnt_ref, g0a_ref, g0b_ref,
             g0c_ref, b0_ref, w1_ref, b1_ref, w2_ref, b2_ref, out_ref):
    ma = aseg_ref[...] / jnp.maximum(acnt_ref[...], 1.0)
    mbo = bseg_ref[...] / jnp.maximum(bcnt_ref[...], 1.0)
    pre = (_dot(gf_ref[...], g0a_ref[...]) + _dot(ma, g0b_ref[...])
           + _dot(mbo, g0c_ref[...]) + b0_ref[...])
    h = _softplus(pre)
    h = _softplus(_dot(h, w1_ref[...]) + b1_ref[...])
    out_ref[...] = _dot(h, w2_ref[...]) + b2_ref[...]


def _full(shape):
    return pl.BlockSpec(shape, lambda i: (0, 0))


def kernel(atom_feats, bond_feats, global_feats, bond_atom, atom_graph,
           bond_graph, atom_w0, atom_b0, atom_w1, atom_b1, atom_w2, atom_b2,
           bond_w0, bond_b0, bond_w1, bond_b1, bond_w2, bond_b2,
           glob_w0, glob_b0, glob_w1, glob_b1, glob_w2, glob_b2):
    f32 = jnp.float32
    e = jnp.concatenate([
        bond_atom.reshape(-1),
        jnp.full((EPAD - 2 * DB,), 2 ** 30, jnp.int32)])
    pad_idx = (jnp.arange(BPAD - DB, dtype=jnp.int32) * 7919) % DA
    src0 = jnp.concatenate([bond_atom[:, 0], pad_idx])
    src1 = jnp.concatenate([bond_atom[:, 1], pad_idx])
    ag2 = atom_graph.reshape(DA, 1)
    bg2 = bond_graph.reshape(DB, 1)

    a0a, a0b, a0g = atom_w0[:D], atom_w0[D:2 * D], atom_w0[2 * D:]
    wb, wa0, wa1, wg = (bond_w0[:D], bond_w0[D:2 * D], bond_w0[2 * D:3 * D],
                        bond_w0[3 * D:])
    g0a, g0b, g0c = glob_w0[:D], glob_w0[D:2 * D], glob_w0[2 * D:]
    ab0, ab1, ab2 = (atom_b0.reshape(1, D), atom_b1.reshape(1, D),
                     atom_b2.reshape(1, D))
    bb0, bb1, bb2 = (bond_b0.reshape(1, D), bond_b1.reshape(1, D),
                     bond_b2.reshape(1, D))
    gb0, gb1, gb2 = (glob_b0.reshape(1, D), glob_b1.reshape(1, D),
                     glob_b2.reshape(1, D))

    gga, ggb = pl.pallas_call(
        _pre_tc,
        out_shape=(jax.ShapeDtypeStruct((DG, D), f32),
                   jax.ShapeDtypeStruct((DG, D), f32)),
    )(global_feats, a0g, wg)

    sum_pad = _scatter_sum(e, bond_feats)
    cnt_pad = _scatter_cnt(e, e)

    wspec = _full((D, D))
    bspec = _full((1, D))
    atom_out, p0, p1, aseg, acnt = pl.pallas_call(
        _atom_tc,
        grid=(DA // BA,),
        in_specs=[
            pl.BlockSpec((BA, D), lambda i: (i, 0)),
            pl.BlockSpec((BA, D), lambda i: (i, 0)),
            pl.BlockSpec((BA, 128), lambda i: (i, 0)),
            pl.BlockSpec((BA, 1), lambda i: (i, 0)),
            wspec, wspec, _full((DG, D)), bspec, wspec, bspec, wspec, bspec,
            wspec, wspec,
        ],
        out_specs=[
            pl.BlockSpec((BA, D), lambda i: (i, 0)),
            pl.BlockSpec((BA, D), lambda i: (i, 0)),
            pl.BlockSpec((BA, D), lambda i: (i, 0)),
            _full((DG, D)),
            _full((DG, D)),
        ],
        out_shape=[
            jax.ShapeDtypeStruct((DA, D), f32),
            jax.ShapeDtypeStruct((DA, D), f32),
            jax.ShapeDtypeStruct((DA, D), f32),
            jax.ShapeDtypeStruct((DG, D), f32),
            jax.ShapeDtypeStruct((DG, D), f32),
        ],
    )(atom_feats, sum_pad, cnt_pad, ag2, a0a, a0b, gga, ab0, atom_w1, ab1,
      atom_w2, ab2, wa0, wa1)

    gs = _gather_pair(src0, src1, p0, p1)

    bond_out, bseg, bcnt = pl.pallas_call(
        _bond_tc,
        grid=(DB // BB,),
        in_specs=[
            pl.BlockSpec((BB, D), lambda i: (i, 0)),
            pl.BlockSpec((BB, D), lambda i: (i, 0)),
            pl.BlockSpec((BB, 1), lambda i: (i, 0)),
            wspec, _full((DG, D)), bspec, wspec, bspec, wspec, bspec,
        ],
        out_specs=[
            pl.BlockSpec((BB, D), lambda i: (i, 0)),
            _full((DG, D)),
            _full((DG, D)),
        ],
        out_shape=[
            jax.ShapeDtypeStruct((DB, D), f32),
            jax.ShapeDtypeStruct((DG, D), f32),
            jax.ShapeDtypeStruct((DG, D), f32),
        ],
    )(bond_feats, gs, bg2, wb, ggb, bb0, bond_w1, bb1, bond_w2, bb2)

    glob_out = pl.pallas_call(
        _glob_tc,
        out_shape=jax.ShapeDtypeStruct((DG, D), f32),
    )(global_feats, aseg, acnt, bseg, bcnt, g0a, g0b, g0c, gb0, glob_w1,
      gb1, glob_w2, gb2)

    return (atom_out, bond_out, glob_out)
